# baseline (speedup 1.0000x reference)
import functools

import jax
import jax.numpy as jnp
from jax import lax
from jax.experimental import pallas as pl
from jax.experimental.pallas import tpu as pltpu
from jax.experimental.pallas import tpu_sc as plsc

VOCAB = 65024
DIM = 16
ROWS = 16384
COLS = 200
B = ROWS * COLS

NC, NS = 2, 16
NW = NC * NS
B_PER_W = B // NW
CHUNK = 200
NCHUNK = B_PER_W // CHUNK
NBUF = 8
GDEPTH = 4
NROUND = NCHUNK // NBUF
STAGE_ROWS = VOCAB // NS
SUBSTAGE = 508
NSTAGE = STAGE_ROWS // SUBSTAGE

_mesh = plsc.VectorSubcoreMesh(core_axis_name="c", subcore_axis_name="s")


@functools.partial(
    pl.kernel,
    out_type=jax.ShapeDtypeStruct((B, DIM), jnp.float32),
    mesh=_mesh,
    scratch_types=[pltpu.VMEM((CHUNK,), jnp.int32)] * NBUF
    + [pltpu.VMEM((CHUNK, DIM), jnp.float32)] * NBUF
    + [pltpu.SemaphoreType.DMA] * (3 * NBUF)
    + [
        pltpu.VMEM_SHARED((VOCAB, DIM), jnp.float32),
        pltpu.VMEM((SUBSTAGE, DIM), jnp.float32),
    ],
    compiler_params=pltpu.CompilerParams(use_tc_tiling_on_sc=False),
)
def _gather_kernel(x_hbm, table_hbm, out_hbm, *scratch):
    idx_v = scratch[0:NBUF]
    rows_v = scratch[NBUF : 2 * NBUF]
    sems = scratch[2 * NBUF : 5 * NBUF]
    sem_i = sems[0:NBUF]
    sem_g = sems[NBUF : 2 * NBUF]
    sem_o = sems[2 * NBUF : 3 * NBUF]
    tab_sh = scratch[5 * NBUF]
    stage_v = scratch[5 * NBUF + 1]

    sid = lax.axis_index("s")
    wid = sid * NC + lax.axis_index("c")
    base = wid * B_PER_W

    row0 = sid * STAGE_ROWS
    for k in range(NSTAGE):
        r0 = row0 + k * SUBSTAGE
        pltpu.sync_copy(table_hbm.at[pl.ds(r0, SUBSTAGE)], stage_v)
        pltpu.sync_copy(stage_v, tab_sh.at[pl.ds(r0, SUBSTAGE)])
    plsc.subcore_barrier()

    def idx_cp(g, b):
        return pltpu.make_async_copy(
            x_hbm.at[pl.ds(base + g * CHUNK, CHUNK)], idx_v[b], sem_i[b]
        )

    def gather_cp(b):
        return pltpu.make_async_copy(tab_sh.at[idx_v[b]], rows_v[b], sem_g[b])

    def store_cp(g, b):
        return pltpu.make_async_copy(
            rows_v[b], out_hbm.at[pl.ds(base + g * CHUNK, CHUNK)], sem_o[b]
        )

    def drain(g, b):
        gq = g - GDEPTH
        bq = gq % NBUF if isinstance(g, int) else (b - GDEPTH) % NBUF
        gather_cp(bq).wait()
        store_cp(gq, bq).start()
        return bq, gq

    for b in range(NBUF):
        idx_cp(b, b).start()
    for g in range(NBUF):
        idx_cp(g, g).wait()
        gather_cp(g).start()
        if g >= GDEPTH:
            bq, gq = drain(g, g)
            idx_cp(gq + NBUF, bq).start()

    def round_body(r, carry):
        g0 = r * NBUF
        for b in range(NBUF):
            g = g0 + b
            idx_cp(g, b).wait()
            store_cp(g - NBUF, b).wait()
            gather_cp(b).start()
            bq = (b - GDEPTH) % NBUF
            gather_cp(bq).wait()
            store_cp(g - GDEPTH, bq).start()
            idx_cp(g - GDEPTH + NBUF, bq).start()
        return carry

    lax.fori_loop(1, NROUND - 1, round_body, 0)

    g0 = (NROUND - 1) * NBUF
    for b in range(NBUF):
        g = g0 + b
        idx_cp(g, b).wait()
        store_cp(g - NBUF, b).wait()
        gather_cp(b).start()
        bq, gq = drain(g, b)
        if gq + NBUF < NCHUNK:
            idx_cp(gq + NBUF, bq).start()

    for g in range(NCHUNK, NCHUNK + GDEPTH):
        drain(g, g % NBUF)
    for b in range(NBUF):
        store_cp(NCHUNK - NBUF + b, (NCHUNK - NBUF + b) % NBUF).wait()


XR_PER_W = ROWS // NW
RBUF = 2
NPAIR = XR_PER_W // RBUF


@functools.partial(
    pl.kernel,
    out_type=jax.ShapeDtypeStruct((ROWS, COLS, DIM), jnp.float32),
    mesh=_mesh,
    scratch_types=[pltpu.VMEM((COLS * DIM,), jnp.float32)] * RBUF
    + [pltpu.VMEM((COLS, DIM), jnp.float32)] * RBUF
    + [pltpu.SemaphoreType.DMA] * (2 * RBUF),
)
def _relayout_kernel(rows_hbm, out_hbm, *scratch):
    in_v = scratch[0:RBUF]
    pad_v = scratch[RBUF : 2 * RBUF]
    sem_i = scratch[2 * RBUF : 3 * RBUF]
    sem_o = scratch[3 * RBUF : 4 * RBUF]

    wid = lax.axis_index("s") * NC + lax.axis_index("c")
    xr0 = wid * XR_PER_W

    def in_cp(i, b):
        return pltpu.make_async_copy(rows_hbm.at[xr0 + i], in_v[b], sem_i[b])

    def out_cp(i, b):
        return pltpu.make_async_copy(pad_v[b], out_hbm.at[xr0 + i], sem_o[b])

    def expand(b):
        for r in range(COLS):
            pad_v[b][r, :] = in_v[b][pl.ds(r * DIM, DIM)]

    for b in range(RBUF):
        in_cp(b, b).start()
    for b in range(RBUF):
        in_cp(b, b).wait()
        expand(b)
        out_cp(b, b).start()
        in_cp(b + RBUF, b).start()

    def pair_body(p, carry):
        for b in range(RBUF):
            ii = p * RBUF + b
            in_cp(ii, b).wait()
            out_cp(ii - RBUF, b).wait()
            expand(b)
            out_cp(ii, b).start()
            in_cp(ii + RBUF, b).start()
        return carry

    lax.fori_loop(1, NPAIR - 1, pair_body, 0)

    for b in range(RBUF):
        ii = (NPAIR - 1) * RBUF + b
        in_cp(ii, b).wait()
        out_cp(ii - RBUF, b).wait()
        expand(b)
        out_cp(ii, b).start()
    for b in range(RBUF):
        out_cp((NPAIR - 1) * RBUF + b, b).wait()


def kernel(X, table):
    flat_idx = X.reshape(B)
    rows = _gather_kernel(flat_idx, table)
    return _relayout_kernel(rows.reshape(ROWS, COLS * DIM))


# speedup vs baseline: 1.0023x; 1.0023x over previous
import functools

import jax
import jax.numpy as jnp
from jax import lax
from jax.experimental import pallas as pl
from jax.experimental.pallas import tpu as pltpu
from jax.experimental.pallas import tpu_sc as plsc

VOCAB = 65024
DIM = 16
ROWS = 16384
COLS = 200
B = ROWS * COLS

NC, NS = 2, 16
NW = NC * NS
B_PER_W = B // NW
CHUNK = 200
NCHUNK = B_PER_W // CHUNK
NBUF = 8
GDEPTH = 4
NROUND = NCHUNK // NBUF
STAGE_ROWS = VOCAB // NS
SUBSTAGE = 508
NSTAGE = STAGE_ROWS // SUBSTAGE

_mesh = plsc.VectorSubcoreMesh(core_axis_name="c", subcore_axis_name="s")


@functools.partial(
    pl.kernel,
    out_type=jax.ShapeDtypeStruct((B, DIM), jnp.float32),
    mesh=_mesh,
    scratch_types=[pltpu.VMEM((CHUNK,), jnp.int32)] * NBUF
    + [pltpu.VMEM((CHUNK, DIM), jnp.float32)] * NBUF
    + [pltpu.SemaphoreType.DMA] * (3 * NBUF)
    + [
        pltpu.VMEM_SHARED((VOCAB, DIM), jnp.float32),
        pltpu.VMEM((SUBSTAGE, DIM), jnp.float32),
    ],
    compiler_params=pltpu.CompilerParams(use_tc_tiling_on_sc=False),
)
def _gather_kernel(x_hbm, table_hbm, out_hbm, *scratch):
    idx_v = scratch[0:NBUF]
    rows_v = scratch[NBUF : 2 * NBUF]
    sems = scratch[2 * NBUF : 5 * NBUF]
    sem_i = sems[0:NBUF]
    sem_g = sems[NBUF : 2 * NBUF]
    sem_o = sems[2 * NBUF : 3 * NBUF]
    tab_sh = scratch[5 * NBUF]
    stage_v = scratch[5 * NBUF + 1]

    sid = lax.axis_index("s")
    wid = sid * NC + lax.axis_index("c")
    base = wid * B_PER_W

    row0 = sid * STAGE_ROWS
    for k in range(NSTAGE):
        r0 = row0 + k * SUBSTAGE
        pltpu.sync_copy(table_hbm.at[pl.ds(r0, SUBSTAGE)], stage_v)
        pltpu.sync_copy(stage_v, tab_sh.at[pl.ds(r0, SUBSTAGE)])
    plsc.subcore_barrier()

    def idx_cp(g, b):
        return pltpu.make_async_copy(
            x_hbm.at[pl.ds(base + g * CHUNK, CHUNK)], idx_v[b], sem_i[b]
        )

    def gather_cp(b):
        return pltpu.make_async_copy(tab_sh.at[idx_v[b]], rows_v[b], sem_g[b])

    def store_cp(g, b):
        return pltpu.make_async_copy(
            rows_v[b], out_hbm.at[pl.ds(base + g * CHUNK, CHUNK)], sem_o[b]
        )

    def drain(g, b):
        gq = g - GDEPTH
        bq = gq % NBUF if isinstance(g, int) else (b - GDEPTH) % NBUF
        gather_cp(bq).wait()
        store_cp(gq, bq).start()
        return bq, gq

    for b in range(NBUF):
        idx_cp(b, b).start()
    for g in range(NBUF):
        idx_cp(g, g).wait()
        gather_cp(g).start()
        if g >= GDEPTH:
            bq, gq = drain(g, g)
            idx_cp(gq + NBUF, bq).start()

    def round_body(r, carry):
        g0 = r * NBUF
        for b in range(NBUF):
            g = g0 + b
            idx_cp(g, b).wait()
            store_cp(g - NBUF, b).wait()
            gather_cp(b).start()
            bq = (b - GDEPTH) % NBUF
            gather_cp(bq).wait()
            store_cp(g - GDEPTH, bq).start()
            idx_cp(g - GDEPTH + NBUF, bq).start()
        return carry

    lax.fori_loop(1, NROUND - 1, round_body, 0)

    g0 = (NROUND - 1) * NBUF
    for b in range(NBUF):
        g = g0 + b
        idx_cp(g, b).wait()
        store_cp(g - NBUF, b).wait()
        gather_cp(b).start()
        bq, gq = drain(g, b)
        if gq + NBUF < NCHUNK:
            idx_cp(gq + NBUF, bq).start()

    for g in range(NCHUNK, NCHUNK + GDEPTH):
        drain(g, g % NBUF)
    for b in range(NBUF):
        store_cp(NCHUNK - NBUF + b, (NCHUNK - NBUF + b) % NBUF).wait()


XR_PER_W = ROWS // NW
RBUF = 2
NPAIR = XR_PER_W // RBUF


@functools.partial(
    pl.kernel,
    out_type=jax.ShapeDtypeStruct((ROWS, COLS, DIM), jnp.float32),
    mesh=_mesh,
    scratch_types=[pltpu.VMEM((COLS * DIM,), jnp.float32)] * RBUF
    + [pltpu.VMEM((COLS, DIM), jnp.float32)] * RBUF
    + [pltpu.SemaphoreType.DMA] * (2 * RBUF),
)
def _relayout_kernel(rows_hbm, out_hbm, *scratch):
    in_v = scratch[0:RBUF]
    pad_v = scratch[RBUF : 2 * RBUF]
    sem_i = scratch[2 * RBUF : 3 * RBUF]
    sem_o = scratch[3 * RBUF : 4 * RBUF]

    wid = lax.axis_index("s") * NC + lax.axis_index("c")
    xr0 = wid * XR_PER_W

    def in_cp(i, b):
        return pltpu.make_async_copy(rows_hbm.at[xr0 + i], in_v[b], sem_i[b])

    def out_cp(i, b):
        return pltpu.make_async_copy(pad_v[b], out_hbm.at[xr0 + i], sem_o[b])

    def expand(b):
        G = 8
        for g in range(COLS // G):
            xs = [in_v[b][pl.ds((g * G + k) * DIM, DIM)] for k in range(G)]
            for k in range(G):
                pad_v[b][g * G + k, :] = xs[k]

    for b in range(RBUF):
        in_cp(b, b).start()
    for b in range(RBUF):
        in_cp(b, b).wait()
        expand(b)
        out_cp(b, b).start()
        in_cp(b + RBUF, b).start()

    def pair_body(p, carry):
        for b in range(RBUF):
            ii = p * RBUF + b
            in_cp(ii, b).wait()
            out_cp(ii - RBUF, b).wait()
            expand(b)
            out_cp(ii, b).start()
            in_cp(ii + RBUF, b).start()
        return carry

    lax.fori_loop(1, NPAIR - 1, pair_body, 0)

    for b in range(RBUF):
        ii = (NPAIR - 1) * RBUF + b
        in_cp(ii, b).wait()
        out_cp(ii - RBUF, b).wait()
        expand(b)
        out_cp(ii, b).start()
    for b in range(RBUF):
        out_cp((NPAIR - 1) * RBUF + b, b).wait()


def kernel(X, table):
    flat_idx = X.reshape(B)
    rows = _gather_kernel(flat_idx, table)
    return _relayout_kernel(rows.reshape(ROWS, COLS * DIM))


# speedup vs baseline: 4.6917x; 4.6808x over previous
import functools

import jax
import jax.numpy as jnp
from jax import lax
from jax.experimental import pallas as pl
from jax.experimental.pallas import tpu as pltpu
from jax.experimental.pallas import tpu_sc as plsc

VOCAB = 65024
DIM = 16
ROWS = 16384
COLS = 200
B = ROWS * COLS

NC, NS = 2, 16
NW = NC * NS
IBLK = 128
NIB = ROWS // IBLK
IB_PER_W = NIB // NW
NBUF = 4
NU = IB_PER_W * COLS

STAGE_ROWS = VOCAB // NS
SUBSTAGE = 508
NSTAGE = STAGE_ROWS // SUBSTAGE

_mesh = plsc.VectorSubcoreMesh(core_axis_name="c", subcore_axis_name="s")


@functools.partial(
    pl.kernel,
    out_type=jax.ShapeDtypeStruct((COLS, 2, NIB, 8, IBLK), jnp.float32),
    mesh=_mesh,
    scratch_types=[pltpu.VMEM((IBLK,), jnp.int32)] * NBUF
    + [pltpu.VMEM((IBLK, DIM), jnp.float32)] * NBUF
    + [pltpu.VMEM((2, 8, IBLK), jnp.float32)] * NBUF
    + [pltpu.SemaphoreType.DMA] * (3 * NBUF)
    + [
        pltpu.VMEM_SHARED((VOCAB, DIM), jnp.float32),
        pltpu.VMEM((SUBSTAGE, DIM), jnp.float32),
    ],
    compiler_params=pltpu.CompilerParams(use_tc_tiling_on_sc=False, needs_layout_passes=False),
)
def _gather_kernel(x_hbm, table_hbm, out_hbm, *scratch):
    idx_v = scratch[0:NBUF]
    rows_v = scratch[NBUF : 2 * NBUF]
    trans_v = scratch[2 * NBUF : 3 * NBUF]
    sems = scratch[3 * NBUF : 6 * NBUF]
    sem_i = sems[0:NBUF]
    sem_g = sems[NBUF : 2 * NBUF]
    sem_o = sems[2 * NBUF : 3 * NBUF]
    tab_sh = scratch[6 * NBUF]
    stage_v = scratch[6 * NBUF + 1]

    sid = lax.axis_index("s")
    wid = sid * NC + lax.axis_index("c")

    row0 = sid * STAGE_ROWS
    for k in range(NSTAGE):
        r0 = row0 + k * SUBSTAGE
        pltpu.sync_copy(table_hbm.at[pl.ds(r0, SUBSTAGE)], stage_v)
        pltpu.sync_copy(stage_v, tab_sh.at[pl.ds(r0, SUBSTAGE)])
    plsc.subcore_barrier()

    row_iota = [
        jax.lax.iota(jnp.int32, DIM) + jnp.int32(i0) for i0 in range(0, IBLK, DIM)
    ]
    dcol = [jnp.full((DIM,), d, jnp.int32) for d in range(DIM)]

    def unit_ib_j(u):
        ib_l = u // COLS
        j = u - ib_l * COLS
        return wid * IB_PER_W + ib_l, j

    def idx_cp(u, b):
        ib, j = unit_ib_j(u)
        return pltpu.make_async_copy(
            x_hbm.at[j, pl.ds(ib * IBLK, IBLK)], idx_v[b], sem_i[b]
        )

    def gather_cp(b):
        return pltpu.make_async_copy(tab_sh.at[idx_v[b]], rows_v[b], sem_g[b])

    def store_cps(u, b):
        ib, j = unit_ib_j(u)
        return [
            pltpu.make_async_copy(
                trans_v[b].at[db], out_hbm.at[j, db, ib], sem_o[b]
            )
            for db in range(2)
        ]

    def transpose(b):
        for g in range(IBLK // DIM):
            segs = [
                plsc.load_gather(rows_v[b], [row_iota[g], dcol[d]])
                for d in range(DIM)
            ]
            for d in range(DIM):
                trans_v[b][d // 8, d % 8, pl.ds(g * DIM, DIM)] = segs[d]

    def step(u, b, do_store_wait, do_prefetch):
        idx_cp(u, b).wait()
        gather_cp(b).start()
        b1 = (b - 1) % NBUF
        gather_cp(b1).wait()
        if do_store_wait:
            for cp in store_cps(u - 1 - NBUF, b1):
                cp.wait()
        transpose(b1)
        for cp in store_cps(u - 1, b1):
            cp.start()
        if do_prefetch:
            idx_cp(u + 2, (b + 2) % NBUF).start()

    idx_cp(0, 0).start()
    idx_cp(1, 1).start()
    idx_cp(0, 0).wait()
    gather_cp(0).start()
    idx_cp(2, 2).start()
    for u in range(1, 8):
        step(u, u % NBUF, do_store_wait=(u >= 5), do_prefetch=True)

    def quad_body(p, carry):
        for k in range(NBUF):
            u = p * NBUF + k
            step(u, k, do_store_wait=True, do_prefetch=True)
        return carry

    lax.fori_loop(2, NU // NBUF - 1, quad_body, 0)

    for u in range(NU - NBUF, NU):
        step(u, u % NBUF, do_store_wait=True, do_prefetch=(u + 2 < NU))
    bl = (NU - 1) % NBUF
    gather_cp(bl).wait()
    for cp in store_cps(NU - 1 - NBUF, bl):
        cp.wait()
    transpose(bl)
    for cp in store_cps(NU - 1, bl):
        cp.start()
    for u in range(NU - NBUF, NU):
        for cp in store_cps(u, u % NBUF):
            cp.wait()


def kernel(X, table):
    phys = _gather_kernel(X.T, table)
    return phys.transpose(2, 4, 0, 1, 3).reshape(ROWS, COLS, DIM)
